# baseline (speedup 1.0000x reference)
import functools

import jax
import jax.numpy as jnp
import numpy as np
from jax import lax
from jax.experimental import pallas as pl
from jax.experimental.pallas import tpu as pltpu
from jax.experimental.pallas import tpu_sc as plsc

NLAT, NLON, KERNEL = 180, 360, 30
C = 64
L = 16
NG = C // L
NC, NS = 2, 16
NW = NC * NS


def _grid_points(H, W):
    lat = np.pi * (np.arange(H) + 0.5) / H - np.pi / 2.0
    lon = 2.0 * np.pi * np.arange(W) / W
    la, lo = np.meshgrid(lat, lon, indexing='ij')
    pts = np.stack([np.cos(la) * np.cos(lo), np.cos(la) * np.sin(lo),
                    np.sin(la)], axis=-1)
    return pts.reshape(H * W, 3)


def _compressed_weights(in_ratio, out_ratio):
    Hi, Wi = int(round(NLAT * in_ratio)), int(round(NLON * in_ratio))
    Ho, Wo = int(round(NLAT * out_ratio)), int(round(NLON * out_ratio))
    P = _grid_points(Hi, Wi)
    M = _grid_points(Ho, Wo).reshape(Ho, Wo, 3)[:, :2]
    ci = np.clip(((np.arange(Ho) + 0.5) * Hi / Ho).astype(np.int64), 0, Hi - 1)
    cj = (((np.arange(2) + 0.5) * Wi / Wo).astype(np.int64)) % Wi
    di = np.array([-2, -1, 0, 1, 2], dtype=np.int64)
    dj = np.array([-3, -2, -1, 0, 1, 2], dtype=np.int64)
    ii = np.clip(ci[:, None, None, None] + di[None, None, :, None], 0, Hi - 1)
    jj = (cj[None, :, None, None] + dj[None, None, None, :]) % Wi
    idx = (ii * Wi + jj).reshape(Ho, 2, KERNEL)
    dots = np.einsum('hpd,hpkd->hpk', M, P[idx])
    kappa = (Hi * Hi) / 4.0
    w = np.exp(kappa * (dots - 1.0))
    w = w / w.sum(axis=2, keepdims=True)
    return w.astype(np.float32)


def _make_layer(in_ratio, out_ratio, max_pairs, tiles_hi):
    Hi, Wi = int(round(NLAT * in_ratio)), int(round(NLON * in_ratio))
    Ho, Wo = int(round(NLAT * out_ratio)), int(round(NLON * out_ratio))
    assert tiles_hi * max_pairs + (NW - tiles_hi) * (max_pairs - 1) == Ho // 2
    n_slots = max_pairs + 4
    BW = Wi + 5
    RW = BW * C
    wrow = 2 * KERNEL * L

    chunks = Wo // 90
    CW = Wo // (2 * chunks)
    UNROLL = 3
    wpair = 2 * wrow

    wc = _compressed_weights(in_ratio, out_ratio)
    wsp = np.repeat(wc.reshape(Ho, 2 * KERNEL), L, axis=1)
    wsp = np.concatenate([wsp, np.zeros((2, wsp.shape[1]), np.float32)])
    wsp = wsp.reshape(-1)

    def layer_body(x_hbm, w_hbm, b_hbm, out_hbm,
                   inbuf, obuf0, obuf1, obuf2, obuf3, wbuf, bbuf,
                   sem_in, sem_out):
        wid = lax.axis_index("s") * NC + lax.axis_index("c")
        npairs = jnp.where(wid < tiles_hi, max_pairs, max_pairs - 1)
        u0 = jnp.where(wid < tiles_hi, max_pairs * wid,
                       max_pairs * tiles_hi + (max_pairs - 1) * (wid - tiles_hi))

        stage = [pltpu.async_copy(b_hbm, bbuf, sem_in),
                 pltpu.async_copy(
                     w_hbm.at[pl.ds(u0 * wpair, max_pairs * wpair)],
                     wbuf, sem_in)]
        for r in range(n_slots):
            i_r = jnp.clip(u0 - 2 + r, 0, Hi - 1)
            src = i_r * (Wi * C)
            dst = r * RW
            stage.append(pltpu.async_copy(
                x_hbm.at[pl.ds(src, Wi * C)],
                inbuf.at[pl.ds(dst + 3 * C, Wi * C)], sem_in))
            stage.append(pltpu.async_copy(
                x_hbm.at[pl.ds(src + (Wi - 3) * C, 3 * C)],
                inbuf.at[pl.ds(dst, 3 * C)], sem_in))
            stage.append(pltpu.async_copy(
                x_hbm.at[pl.ds(src, 2 * C)],
                inbuf.at[pl.ds(dst + (3 + Wi) * C, 2 * C)], sem_in))
        for h in stage:
            h.wait()

        bv = [bbuf[pl.ds(L * g, L)] for g in range(NG)]
        obufs = [obuf0, obuf1, obuf2, obuf3]

        def row_pair(u_rel, carry):
            @pl.when(u_rel < npairs)
            def _():
                out_dmas = []
                ho = 2 * u0 + 2 * u_rel
                wb = [(2 * u_rel + rloc) * wrow for rloc in range(2)]
                for ck in range(chunks):
                    if ck >= 2:
                        out_dmas[2 * (ck - 2)].wait()
                        out_dmas[2 * (ck - 2) + 1].wait()
                    obs = [obufs[2 * (ck % 2)], obufs[2 * (ck % 2) + 1]]
                    cbase = u_rel * RW + ck * (CW * C)
                    for a in range(5):
                        sbase = cbase + a * RW
                        wv = [[[wbuf[pl.ds(wb[r] + (p * KERNEL + a * 6 + b) * L, L)]
                                for b in range(4)] for p in range(2)]
                              for r in range(2)]

                        def jbody(jj, c2, a=a, sbase=sbase, wv=wv, obs=obs):
                            for up in range(UNROLL):
                                jp = jj * UNROLL + up
                                off = sbase + jp * C
                                ab = 2 * jp * C
                                for g in range(NG):
                                    v = [inbuf[pl.ds(off + b * C + L * g, L)]
                                         for b in range(6)]
                                    s01 = v[1] + v[5]
                                    s02 = v[2] + v[4]
                                    s12 = v[2] + v[5]
                                    s13 = v[3] + v[4]
                                    for r in range(2):
                                        ob = obs[r]
                                        w0, w1 = wv[r]
                                        if a == 0:
                                            a0 = bv[g]
                                            a1 = bv[g]
                                        else:
                                            a0 = ob[pl.ds(ab + L * g, L)]
                                            a1 = ob[pl.ds(ab + C + L * g, L)]
                                        a0 = (a0 + v[0] * w0[0] + s01 * w0[1]
                                              + s02 * w0[2] + v[3] * w0[3])
                                        a1 = (a1 + v[0] * w1[0] + v[1] * w1[1]
                                              + s12 * w1[2] + s13 * w1[3])
                                        ob[pl.ds(ab + L * g, L)] = a0
                                        ob[pl.ds(ab + C + L * g, L)] = a1
                            return c2

                        lax.fori_loop(0, CW // UNROLL, jbody, 0)
                    for r in range(2):
                        out_dmas.append(pltpu.async_copy(
                            obs[r],
                            out_hbm.at[pl.ds((ho + r) * (Wo * C)
                                             + ck * (2 * CW * C),
                                             2 * CW * C)],
                            sem_out))
                for d in out_dmas[max(0, 2 * (chunks - 2)):]:
                    d.wait()
            return carry

        lax.fori_loop(0, max_pairs, row_pair, 0)

    built = []

    def apply(x, bias):
        if not built:
            mesh = plsc.VectorSubcoreMesh(
                core_axis_name="c", subcore_axis_name="s",
                num_cores=NC, num_subcores=NS)
            built.append(pl.kernel(
                layer_body,
                out_type=jax.ShapeDtypeStruct((Ho * Wo * C,), jnp.float32),
                mesh=mesh,
                scratch_types=[
                    pltpu.VMEM((n_slots * RW,), jnp.float32),
                    pltpu.VMEM((2 * CW * C,), jnp.float32),
                    pltpu.VMEM((2 * CW * C,), jnp.float32),
                    pltpu.VMEM((2 * CW * C,), jnp.float32),
                    pltpu.VMEM((2 * CW * C,), jnp.float32),
                    pltpu.VMEM((max_pairs * wpair,), jnp.float32),
                    pltpu.VMEM((C,), jnp.float32),
                    pltpu.SemaphoreType.DMA,
                    pltpu.SemaphoreType.DMA,
                ],
            ))
        return built[0](x, jnp.asarray(wsp), bias)

    return apply


_layer1 = _make_layer(0.25, 0.5, 2, 13)
_layer2 = _make_layer(0.5, 1.0, 3, 26)


@jax.jit
def kernel(x, b1, b2):
    xf = x.reshape(-1)
    h = _layer1(xf, b1)
    y = _layer2(h, b2)
    return y.reshape(1, NLAT * NLON, C)


# speedup vs baseline: 1.0821x; 1.0821x over previous
import functools

import jax
import jax.numpy as jnp
import numpy as np
from jax import lax
from jax.experimental import pallas as pl
from jax.experimental.pallas import tpu as pltpu
from jax.experimental.pallas import tpu_sc as plsc

NLAT, NLON, KERNEL = 180, 360, 30
C = 64
L = 16
NG = C // L
NC, NS = 2, 16
NW = NC * NS


def _grid_points(H, W):
    lat = np.pi * (np.arange(H) + 0.5) / H - np.pi / 2.0
    lon = 2.0 * np.pi * np.arange(W) / W
    la, lo = np.meshgrid(lat, lon, indexing='ij')
    pts = np.stack([np.cos(la) * np.cos(lo), np.cos(la) * np.sin(lo),
                    np.sin(la)], axis=-1)
    return pts.reshape(H * W, 3)


def _compressed_weights(in_ratio, out_ratio):
    Hi, Wi = int(round(NLAT * in_ratio)), int(round(NLON * in_ratio))
    Ho, Wo = int(round(NLAT * out_ratio)), int(round(NLON * out_ratio))
    P = _grid_points(Hi, Wi)
    M = _grid_points(Ho, Wo).reshape(Ho, Wo, 3)[:, :2]
    ci = np.clip(((np.arange(Ho) + 0.5) * Hi / Ho).astype(np.int64), 0, Hi - 1)
    cj = (((np.arange(2) + 0.5) * Wi / Wo).astype(np.int64)) % Wi
    di = np.array([-2, -1, 0, 1, 2], dtype=np.int64)
    dj = np.array([-3, -2, -1, 0, 1, 2], dtype=np.int64)
    ii = np.clip(ci[:, None, None, None] + di[None, None, :, None], 0, Hi - 1)
    jj = (cj[None, :, None, None] + dj[None, None, None, :]) % Wi
    idx = (ii * Wi + jj).reshape(Ho, 2, KERNEL)
    dots = np.einsum('hpd,hpkd->hpk', M, P[idx])
    kappa = (Hi * Hi) / 4.0
    w = np.exp(kappa * (dots - 1.0))
    w = w / w.sum(axis=2, keepdims=True)
    return w.astype(np.float32)


def _make_layer(in_ratio, out_ratio, max_pairs, tiles_hi):
    Hi, Wi = int(round(NLAT * in_ratio)), int(round(NLON * in_ratio))
    Ho, Wo = int(round(NLAT * out_ratio)), int(round(NLON * out_ratio))
    assert tiles_hi * max_pairs + (NW - tiles_hi) * (max_pairs - 1) == Ho // 2
    n_slots = max_pairs + 4
    BW = Wi + 5
    RW = BW * C
    wrow = 2 * KERNEL * L

    chunks = Wo // 90
    CW = Wo // (2 * chunks)
    UNROLL = 1
    wpair = 2 * wrow

    wc = _compressed_weights(in_ratio, out_ratio)
    wsp = np.repeat(wc.reshape(Ho, 2 * KERNEL), L, axis=1)
    wsp = np.concatenate([wsp, np.zeros((2, wsp.shape[1]), np.float32)])
    wsp = wsp.reshape(-1)

    def layer_body(x_hbm, w_hbm, b_hbm, out_hbm,
                   inbuf, obuf0, obuf1, obuf2, obuf3, wbuf, bbuf,
                   sem_in, sem_out):
        wid = lax.axis_index("s") * NC + lax.axis_index("c")
        npairs = jnp.where(wid < tiles_hi, max_pairs, max_pairs - 1)
        u0 = jnp.where(wid < tiles_hi, max_pairs * wid,
                       max_pairs * tiles_hi + (max_pairs - 1) * (wid - tiles_hi))

        stage = [pltpu.async_copy(b_hbm, bbuf, sem_in),
                 pltpu.async_copy(
                     w_hbm.at[pl.ds(u0 * wpair, max_pairs * wpair)],
                     wbuf, sem_in)]
        for r in range(n_slots):
            i_r = jnp.clip(u0 - 2 + r, 0, Hi - 1)
            src = i_r * (Wi * C)
            dst = r * RW
            stage.append(pltpu.async_copy(
                x_hbm.at[pl.ds(src, Wi * C)],
                inbuf.at[pl.ds(dst + 3 * C, Wi * C)], sem_in))
            stage.append(pltpu.async_copy(
                x_hbm.at[pl.ds(src + (Wi - 3) * C, 3 * C)],
                inbuf.at[pl.ds(dst, 3 * C)], sem_in))
            stage.append(pltpu.async_copy(
                x_hbm.at[pl.ds(src, 2 * C)],
                inbuf.at[pl.ds(dst + (3 + Wi) * C, 2 * C)], sem_in))
        for h in stage:
            h.wait()

        bv = [bbuf[pl.ds(L * g, L)] for g in range(NG)]
        obufs = [obuf0, obuf1, obuf2, obuf3]

        def row_pair(u_rel, carry):
            @pl.when(u_rel < npairs)
            def _():
                out_dmas = []
                ho = 2 * u0 + 2 * u_rel
                wb = [(2 * u_rel + rloc) * wrow for rloc in range(2)]
                for ck in range(chunks):
                    if ck >= 2:
                        out_dmas[2 * (ck - 2)].wait()
                        out_dmas[2 * (ck - 2) + 1].wait()
                    obs = [obufs[2 * (ck % 2)], obufs[2 * (ck % 2) + 1]]
                    cbase = u_rel * RW + ck * (CW * C)
                    for a in range(5):
                        sbase = cbase + a * RW
                        wv = [[[wbuf[pl.ds(wb[r] + (p * KERNEL + a * 6 + b) * L, L)]
                                for b in range(4)] for p in range(2)]
                              for r in range(2)]

                        def jbody(jj, c2, a=a, sbase=sbase, wv=wv, obs=obs):
                            for up in range(UNROLL):
                                jp = jj * UNROLL + up
                                off = sbase + jp * C
                                ab = 2 * jp * C
                                for g in range(NG):
                                    v = [inbuf[pl.ds(off + b * C + L * g, L)]
                                         for b in range(6)]
                                    s01 = v[1] + v[5]
                                    s02 = v[2] + v[4]
                                    s12 = v[2] + v[5]
                                    s13 = v[3] + v[4]
                                    for r in range(2):
                                        ob = obs[r]
                                        w0, w1 = wv[r]
                                        if a == 0:
                                            a0 = bv[g]
                                            a1 = bv[g]
                                        else:
                                            a0 = ob[pl.ds(ab + L * g, L)]
                                            a1 = ob[pl.ds(ab + C + L * g, L)]
                                        a0 = (a0 + v[0] * w0[0] + s01 * w0[1]
                                              + s02 * w0[2] + v[3] * w0[3])
                                        a1 = (a1 + v[0] * w1[0] + v[1] * w1[1]
                                              + s12 * w1[2] + s13 * w1[3])
                                        ob[pl.ds(ab + L * g, L)] = a0
                                        ob[pl.ds(ab + C + L * g, L)] = a1
                            return c2

                        lax.fori_loop(0, CW // UNROLL, jbody, 0)
                    for r in range(2):
                        out_dmas.append(pltpu.async_copy(
                            obs[r],
                            out_hbm.at[pl.ds((ho + r) * (Wo * C)
                                             + ck * (2 * CW * C),
                                             2 * CW * C)],
                            sem_out))
                for d in out_dmas[max(0, 2 * (chunks - 2)):]:
                    d.wait()
            return carry

        lax.fori_loop(0, max_pairs, row_pair, 0)

    built = []

    def apply(x, bias):
        if not built:
            mesh = plsc.VectorSubcoreMesh(
                core_axis_name="c", subcore_axis_name="s",
                num_cores=NC, num_subcores=NS)
            built.append(pl.kernel(
                layer_body,
                out_type=jax.ShapeDtypeStruct((Ho * Wo * C,), jnp.float32),
                mesh=mesh,
                scratch_types=[
                    pltpu.VMEM((n_slots * RW,), jnp.float32),
                    pltpu.VMEM((2 * CW * C,), jnp.float32),
                    pltpu.VMEM((2 * CW * C,), jnp.float32),
                    pltpu.VMEM((2 * CW * C,), jnp.float32),
                    pltpu.VMEM((2 * CW * C,), jnp.float32),
                    pltpu.VMEM((max_pairs * wpair,), jnp.float32),
                    pltpu.VMEM((C,), jnp.float32),
                    pltpu.SemaphoreType.DMA,
                    pltpu.SemaphoreType.DMA,
                ],
            ))
        return built[0](x, jnp.asarray(wsp), bias)

    return apply


_layer1 = _make_layer(0.25, 0.5, 2, 13)
_layer2 = _make_layer(0.5, 1.0, 3, 26)


@jax.jit
def kernel(x, b1, b2):
    xf = x.reshape(-1)
    h = _layer1(xf, b1)
    y = _layer2(h, b2)
    return y.reshape(1, NLAT * NLON, C)


# speedup vs baseline: 1.3306x; 1.2297x over previous
import functools

import jax
import jax.numpy as jnp
import numpy as np
from jax import lax
from jax.experimental import pallas as pl
from jax.experimental.pallas import tpu as pltpu
from jax.experimental.pallas import tpu_sc as plsc

NLAT, NLON, KERNEL = 180, 360, 30
C = 64
L = 16
NG = C // L
NC, NS = 2, 16
NW = NC * NS


def _grid_points(H, W):
    lat = np.pi * (np.arange(H) + 0.5) / H - np.pi / 2.0
    lon = 2.0 * np.pi * np.arange(W) / W
    la, lo = np.meshgrid(lat, lon, indexing='ij')
    pts = np.stack([np.cos(la) * np.cos(lo), np.cos(la) * np.sin(lo),
                    np.sin(la)], axis=-1)
    return pts.reshape(H * W, 3)


def _compressed_weights(in_ratio, out_ratio):
    Hi, Wi = int(round(NLAT * in_ratio)), int(round(NLON * in_ratio))
    Ho, Wo = int(round(NLAT * out_ratio)), int(round(NLON * out_ratio))
    P = _grid_points(Hi, Wi)
    M = _grid_points(Ho, Wo).reshape(Ho, Wo, 3)[:, :2]
    ci = np.clip(((np.arange(Ho) + 0.5) * Hi / Ho).astype(np.int64), 0, Hi - 1)
    cj = (((np.arange(2) + 0.5) * Wi / Wo).astype(np.int64)) % Wi
    di = np.array([-2, -1, 0, 1, 2], dtype=np.int64)
    dj = np.array([-3, -2, -1, 0, 1, 2], dtype=np.int64)
    ii = np.clip(ci[:, None, None, None] + di[None, None, :, None], 0, Hi - 1)
    jj = (cj[None, :, None, None] + dj[None, None, None, :]) % Wi
    idx = (ii * Wi + jj).reshape(Ho, 2, KERNEL)
    dots = np.einsum('hpd,hpkd->hpk', M, P[idx])
    kappa = (Hi * Hi) / 4.0
    w = np.exp(kappa * (dots - 1.0))
    w = w / w.sum(axis=2, keepdims=True)
    return w.astype(np.float32)


def _make_layer(in_ratio, out_ratio, max_pairs, tiles_hi):
    Hi, Wi = int(round(NLAT * in_ratio)), int(round(NLON * in_ratio))
    Ho, Wo = int(round(NLAT * out_ratio)), int(round(NLON * out_ratio))
    assert tiles_hi * max_pairs + (NW - tiles_hi) * (max_pairs - 1) == Ho // 2
    n_slots = max_pairs + 4
    BW = Wi + 5
    RW = BW * C
    wrow = 2 * KERNEL * L

    chunks = Wo // 180
    CW = Wo // (2 * chunks)
    UNROLL = 1
    wpair = 2 * wrow

    wc = _compressed_weights(in_ratio, out_ratio)
    wsp = np.repeat(wc.reshape(Ho, 2 * KERNEL), L, axis=1)
    wsp = np.concatenate([wsp, np.zeros((2, wsp.shape[1]), np.float32)])
    wsp = wsp.reshape(-1)

    def layer_body(x_hbm, w_hbm, b_hbm, out_hbm,
                   inbuf, obuf0, obuf1, wbuf, bbuf, sem_in, sem_out):
        wid = lax.axis_index("s") * NC + lax.axis_index("c")
        npairs = jnp.where(wid < tiles_hi, max_pairs, max_pairs - 1)
        u0 = jnp.where(wid < tiles_hi, max_pairs * wid,
                       max_pairs * tiles_hi + (max_pairs - 1) * (wid - tiles_hi))

        stage = [pltpu.async_copy(b_hbm, bbuf, sem_in),
                 pltpu.async_copy(
                     w_hbm.at[pl.ds(u0 * wpair, max_pairs * wpair)],
                     wbuf, sem_in)]
        for r in range(n_slots):
            i_r = jnp.clip(u0 - 2 + r, 0, Hi - 1)
            src = i_r * (Wi * C)
            dst = r * RW
            stage.append(pltpu.async_copy(
                x_hbm.at[pl.ds(src, Wi * C)],
                inbuf.at[pl.ds(dst + 3 * C, Wi * C)], sem_in))
            stage.append(pltpu.async_copy(
                x_hbm.at[pl.ds(src + (Wi - 3) * C, 3 * C)],
                inbuf.at[pl.ds(dst, 3 * C)], sem_in))
            stage.append(pltpu.async_copy(
                x_hbm.at[pl.ds(src, 2 * C)],
                inbuf.at[pl.ds(dst + (3 + Wi) * C, 2 * C)], sem_in))
        for h in stage:
            h.wait()

        bv = [bbuf[pl.ds(L * g, L)] for g in range(NG)]
        obufs = [obuf0, obuf1]

        def row_pair(u_rel, carry):
            @pl.when(u_rel < npairs)
            def _():
                out_dmas = []
                for rloc in range(2):
                    wbase = (2 * u_rel + rloc) * wrow
                    ho = 2 * u0 + 2 * u_rel + rloc
                    for ck in range(chunks):
                        k = rloc * chunks + ck
                        if k >= 2:
                            out_dmas[k - 2].wait()
                        ob = obufs[k % 2]
                        cbase = (u_rel) * RW + ck * (CW * C)
                        for a in range(5):
                            sbase = cbase + a * RW
                            wv = [[wbuf[pl.ds(wbase + (p * KERNEL + a * 6 + b) * L, L)]
                                   for b in range(4)] for p in range(2)]

                            def jbody(jj, c2, a=a, sbase=sbase, wv=wv, ob=ob):
                                for up in range(UNROLL):
                                    jp = jj * UNROLL + up
                                    off = sbase + jp * C
                                    ab = 2 * jp * C
                                    if a == 0:
                                        acc0 = list(bv)
                                        acc1 = list(bv)
                                    else:
                                        acc0 = [ob[pl.ds(ab + L * g, L)]
                                                for g in range(NG)]
                                        acc1 = [ob[pl.ds(ab + C + L * g, L)]
                                                for g in range(NG)]
                                    for g in range(NG):
                                        v = [inbuf[pl.ds(off + b * C + L * g, L)]
                                             for b in range(6)]
                                        acc0[g] = (acc0[g] + v[0] * wv[0][0]
                                                   + (v[1] + v[5]) * wv[0][1]
                                                   + (v[2] + v[4]) * wv[0][2]
                                                   + v[3] * wv[0][3])
                                        acc1[g] = (acc1[g] + v[0] * wv[1][0]
                                                   + v[1] * wv[1][1]
                                                   + (v[2] + v[5]) * wv[1][2]
                                                   + (v[3] + v[4]) * wv[1][3])
                                    for g in range(NG):
                                        ob[pl.ds(ab + L * g, L)] = acc0[g]
                                        ob[pl.ds(ab + C + L * g, L)] = acc1[g]
                                return c2

                            lax.fori_loop(0, CW // UNROLL, jbody, 0)
                        out_dmas.append(pltpu.async_copy(
                            ob,
                            out_hbm.at[pl.ds(ho * (Wo * C) + ck * (2 * CW * C),
                                             2 * CW * C)],
                            sem_out))
                for d in out_dmas[-2:]:
                    d.wait()
            return carry

        lax.fori_loop(0, max_pairs, row_pair, 0)

    built = []

    def apply(x, bias):
        if not built:
            mesh = plsc.VectorSubcoreMesh(
                core_axis_name="c", subcore_axis_name="s",
                num_cores=NC, num_subcores=NS)
            built.append(pl.kernel(
                layer_body,
                out_type=jax.ShapeDtypeStruct((Ho * Wo * C,), jnp.float32),
                mesh=mesh,
                scratch_types=[
                    pltpu.VMEM((n_slots * RW,), jnp.float32),
                    pltpu.VMEM((2 * CW * C,), jnp.float32),
                    pltpu.VMEM((2 * CW * C,), jnp.float32),
                    pltpu.VMEM((max_pairs * wpair,), jnp.float32),
                    pltpu.VMEM((C,), jnp.float32),
                    pltpu.SemaphoreType.DMA,
                    pltpu.SemaphoreType.DMA,
                ],
            ))
        return built[0](x, jnp.asarray(wsp), bias)

    return apply


_layer1 = _make_layer(0.25, 0.5, 2, 13)
_layer2 = _make_layer(0.5, 1.0, 3, 26)


@jax.jit
def kernel(x, b1, b2):
    xf = x.reshape(-1)
    h = _layer1(xf, b1)
    y = _layer2(h, b2)
    return y.reshape(1, NLAT * NLON, C)


# speedup vs baseline: 1.6343x; 1.2282x over previous
import functools

import jax
import jax.numpy as jnp
import numpy as np
from jax import lax
from jax.experimental import pallas as pl
from jax.experimental.pallas import tpu as pltpu
from jax.experimental.pallas import tpu_sc as plsc

NLAT, NLON, KERNEL = 180, 360, 30
C = 64
L = 16
NG = C // L
NC, NS = 2, 16
NW = NC * NS


def _grid_points(H, W):
    lat = np.pi * (np.arange(H) + 0.5) / H - np.pi / 2.0
    lon = 2.0 * np.pi * np.arange(W) / W
    la, lo = np.meshgrid(lat, lon, indexing='ij')
    pts = np.stack([np.cos(la) * np.cos(lo), np.cos(la) * np.sin(lo),
                    np.sin(la)], axis=-1)
    return pts.reshape(H * W, 3)


def _compressed_weights(in_ratio, out_ratio):
    Hi, Wi = int(round(NLAT * in_ratio)), int(round(NLON * in_ratio))
    Ho, Wo = int(round(NLAT * out_ratio)), int(round(NLON * out_ratio))
    P = _grid_points(Hi, Wi)
    M = _grid_points(Ho, Wo).reshape(Ho, Wo, 3)[:, :2]
    ci = np.clip(((np.arange(Ho) + 0.5) * Hi / Ho).astype(np.int64), 0, Hi - 1)
    cj = (((np.arange(2) + 0.5) * Wi / Wo).astype(np.int64)) % Wi
    di = np.array([-2, -1, 0, 1, 2], dtype=np.int64)
    dj = np.array([-3, -2, -1, 0, 1, 2], dtype=np.int64)
    ii = np.clip(ci[:, None, None, None] + di[None, None, :, None], 0, Hi - 1)
    jj = (cj[None, :, None, None] + dj[None, None, None, :]) % Wi
    idx = (ii * Wi + jj).reshape(Ho, 2, KERNEL)
    dots = np.einsum('hpd,hpkd->hpk', M, P[idx])
    kappa = (Hi * Hi) / 4.0
    w = np.exp(kappa * (dots - 1.0))
    w = w / w.sum(axis=2, keepdims=True)
    return w.astype(np.float32)


def _make_layer(in_ratio, out_ratio, max_pairs, tiles_hi):
    Hi, Wi = int(round(NLAT * in_ratio)), int(round(NLON * in_ratio))
    Ho, Wo = int(round(NLAT * out_ratio)), int(round(NLON * out_ratio))
    assert tiles_hi * max_pairs + (NW - tiles_hi) * (max_pairs - 1) == Ho // 2
    n_slots = max_pairs + 4
    BW = Wi + 5
    RW = BW * C
    wrow = 2 * KERNEL * L

    chunks = Wo // 180
    CW = Wo // (2 * chunks)
    UNROLL = 1
    wpair = 2 * wrow

    wc = _compressed_weights(in_ratio, out_ratio)
    wsp = np.repeat(wc.reshape(Ho, 2 * KERNEL), L, axis=1)
    wsp = np.concatenate([wsp, np.zeros((2, wsp.shape[1]), np.float32)])
    wsp = wsp.reshape(-1)

    def layer_body(x_hbm, w_hbm, b_hbm, out_hbm,
                   inbuf, obuf0, obuf1, wbuf, bbuf, sem_in, sem_out):
        wid = lax.axis_index("s") * NC + lax.axis_index("c")
        npairs = jnp.where(wid < tiles_hi, max_pairs, max_pairs - 1)
        u0 = jnp.where(wid < tiles_hi, max_pairs * wid,
                       max_pairs * tiles_hi + (max_pairs - 1) * (wid - tiles_hi))

        stage = [pltpu.async_copy(b_hbm, bbuf, sem_in),
                 pltpu.async_copy(
                     w_hbm.at[pl.ds(u0 * wpair, max_pairs * wpair)],
                     wbuf, sem_in)]
        for r in range(n_slots):
            i_r = jnp.clip(u0 - 2 + r, 0, Hi - 1)
            src = i_r * (Wi * C)
            dst = r * RW
            stage.append(pltpu.async_copy(
                x_hbm.at[pl.ds(src, Wi * C)],
                inbuf.at[pl.ds(dst + 3 * C, Wi * C)], sem_in))
            stage.append(pltpu.async_copy(
                x_hbm.at[pl.ds(src + (Wi - 3) * C, 3 * C)],
                inbuf.at[pl.ds(dst, 3 * C)], sem_in))
            stage.append(pltpu.async_copy(
                x_hbm.at[pl.ds(src, 2 * C)],
                inbuf.at[pl.ds(dst + (3 + Wi) * C, 2 * C)], sem_in))
        for h in stage:
            h.wait()

        bv = [bbuf[pl.ds(L * g, L)] for g in range(NG)]
        obufs = [obuf0, obuf1]

        def row_pair(u_rel, carry):
            @pl.when(u_rel < npairs)
            def _():
                out_dmas = []
                for rloc in range(2):
                    wbase = (2 * u_rel + rloc) * wrow
                    ho = 2 * u0 + 2 * u_rel + rloc
                    for ck in range(chunks):
                        k = rloc * chunks + ck
                        if k >= 2:
                            out_dmas[k - 2].wait()
                        ob = obufs[k % 2]
                        cbase = (u_rel) * RW + ck * (CW * C)

                        def jbody(jj, c2, ob=ob):
                            for up in range(UNROLL):
                                jp = jj * UNROLL + up
                                ab = 2 * jp * C
                                acc0 = list(bv)
                                acc1 = list(bv)
                                for a in range(5):
                                    off = cbase + a * RW + jp * C
                                    wv = [[wbuf[pl.ds(
                                        wbase + (p * KERNEL + a * 6 + b) * L, L)]
                                        for b in range(4)] for p in range(2)]
                                    for g in range(NG):
                                        v = [inbuf[pl.ds(off + b * C + L * g, L)]
                                             for b in range(6)]
                                        acc0[g] = (acc0[g] + v[0] * wv[0][0]
                                                   + (v[1] + v[5]) * wv[0][1]
                                                   + (v[2] + v[4]) * wv[0][2]
                                                   + v[3] * wv[0][3])
                                        acc1[g] = (acc1[g] + v[0] * wv[1][0]
                                                   + v[1] * wv[1][1]
                                                   + (v[2] + v[5]) * wv[1][2]
                                                   + (v[3] + v[4]) * wv[1][3])
                                for g in range(NG):
                                    ob[pl.ds(ab + L * g, L)] = acc0[g]
                                    ob[pl.ds(ab + C + L * g, L)] = acc1[g]
                            return c2

                        lax.fori_loop(0, CW // UNROLL, jbody, 0)
                        out_dmas.append(pltpu.async_copy(
                            ob,
                            out_hbm.at[pl.ds(ho * (Wo * C) + ck * (2 * CW * C),
                                             2 * CW * C)],
                            sem_out))
                for d in out_dmas[-2:]:
                    d.wait()
            return carry

        lax.fori_loop(0, max_pairs, row_pair, 0)

    built = []

    def apply(x, bias):
        if not built:
            mesh = plsc.VectorSubcoreMesh(
                core_axis_name="c", subcore_axis_name="s",
                num_cores=NC, num_subcores=NS)
            built.append(pl.kernel(
                layer_body,
                out_type=jax.ShapeDtypeStruct((Ho * Wo * C,), jnp.float32),
                mesh=mesh,
                scratch_types=[
                    pltpu.VMEM((n_slots * RW,), jnp.float32),
                    pltpu.VMEM((2 * CW * C,), jnp.float32),
                    pltpu.VMEM((2 * CW * C,), jnp.float32),
                    pltpu.VMEM((max_pairs * wpair,), jnp.float32),
                    pltpu.VMEM((C,), jnp.float32),
                    pltpu.SemaphoreType.DMA,
                    pltpu.SemaphoreType.DMA,
                ],
            ))
        return built[0](x, jnp.asarray(wsp), bias)

    return apply


_layer1 = _make_layer(0.25, 0.5, 2, 13)
_layer2 = _make_layer(0.5, 1.0, 3, 26)


@jax.jit
def kernel(x, b1, b2):
    xf = x.reshape(-1)
    h = _layer1(xf, b1)
    y = _layer2(h, b2)
    return y.reshape(1, NLAT * NLON, C)


# speedup vs baseline: 1.7869x; 1.0934x over previous
import functools

import jax
import jax.numpy as jnp
import numpy as np
from jax import lax
from jax.experimental import pallas as pl
from jax.experimental.pallas import tpu as pltpu
from jax.experimental.pallas import tpu_sc as plsc

NLAT, NLON, KERNEL = 180, 360, 30
C = 64
L = 16
NG = C // L
NC, NS = 2, 16
NW = NC * NS


def _grid_points(H, W):
    lat = np.pi * (np.arange(H) + 0.5) / H - np.pi / 2.0
    lon = 2.0 * np.pi * np.arange(W) / W
    la, lo = np.meshgrid(lat, lon, indexing='ij')
    pts = np.stack([np.cos(la) * np.cos(lo), np.cos(la) * np.sin(lo),
                    np.sin(la)], axis=-1)
    return pts.reshape(H * W, 3)


def _compressed_weights(in_ratio, out_ratio):
    Hi, Wi = int(round(NLAT * in_ratio)), int(round(NLON * in_ratio))
    Ho, Wo = int(round(NLAT * out_ratio)), int(round(NLON * out_ratio))
    P = _grid_points(Hi, Wi)
    M = _grid_points(Ho, Wo).reshape(Ho, Wo, 3)[:, :2]
    ci = np.clip(((np.arange(Ho) + 0.5) * Hi / Ho).astype(np.int64), 0, Hi - 1)
    cj = (((np.arange(2) + 0.5) * Wi / Wo).astype(np.int64)) % Wi
    di = np.array([-2, -1, 0, 1, 2], dtype=np.int64)
    dj = np.array([-3, -2, -1, 0, 1, 2], dtype=np.int64)
    ii = np.clip(ci[:, None, None, None] + di[None, None, :, None], 0, Hi - 1)
    jj = (cj[None, :, None, None] + dj[None, None, None, :]) % Wi
    idx = (ii * Wi + jj).reshape(Ho, 2, KERNEL)
    dots = np.einsum('hpd,hpkd->hpk', M, P[idx])
    kappa = (Hi * Hi) / 4.0
    w = np.exp(kappa * (dots - 1.0))
    w = w / w.sum(axis=2, keepdims=True)
    return w.astype(np.float32)


def _make_layer(in_ratio, out_ratio, max_pairs, tiles_hi):
    Hi, Wi = int(round(NLAT * in_ratio)), int(round(NLON * in_ratio))
    Ho, Wo = int(round(NLAT * out_ratio)), int(round(NLON * out_ratio))
    assert tiles_hi * max_pairs + (NW - tiles_hi) * (max_pairs - 1) == Ho // 2
    n_slots = max_pairs + 4
    BW = Wi + 5
    RW = BW * C
    wrow = 2 * KERNEL * L

    chunks = Wo // 180
    CW = Wo // (2 * chunks)
    UNROLL = 1
    wpair = 2 * wrow

    wc = _compressed_weights(in_ratio, out_ratio)
    wsp = np.repeat(wc.reshape(Ho, 2 * KERNEL), L, axis=1)
    wsp = np.concatenate([wsp, np.zeros((2, wsp.shape[1]), np.float32)])
    wsp = wsp.reshape(-1)

    def layer_body(x_hbm, w_hbm, b_hbm, out_hbm,
                   inbuf, obuf0, obuf1, wbuf, bbuf, sem_in, sem_out):
        wid = lax.axis_index("s") * NC + lax.axis_index("c")
        npairs = jnp.where(wid < tiles_hi, max_pairs, max_pairs - 1)
        u0 = jnp.where(wid < tiles_hi, max_pairs * wid,
                       max_pairs * tiles_hi + (max_pairs - 1) * (wid - tiles_hi))

        stage = [pltpu.async_copy(b_hbm, bbuf, sem_in),
                 pltpu.async_copy(
                     w_hbm.at[pl.ds(u0 * wpair, max_pairs * wpair)],
                     wbuf, sem_in)]
        for r in range(n_slots):
            i_r = jnp.clip(u0 - 2 + r, 0, Hi - 1)
            src = i_r * (Wi * C)
            dst = r * RW
            stage.append(pltpu.async_copy(
                x_hbm.at[pl.ds(src, Wi * C)],
                inbuf.at[pl.ds(dst + 3 * C, Wi * C)], sem_in))
            stage.append(pltpu.async_copy(
                x_hbm.at[pl.ds(src + (Wi - 3) * C, 3 * C)],
                inbuf.at[pl.ds(dst, 3 * C)], sem_in))
            stage.append(pltpu.async_copy(
                x_hbm.at[pl.ds(src, 2 * C)],
                inbuf.at[pl.ds(dst + (3 + Wi) * C, 2 * C)], sem_in))
        for h in stage:
            h.wait()

        bv = [bbuf[pl.ds(L * g, L)] for g in range(NG)]
        obufs = [obuf0, obuf1]

        def row_pair(u_rel, carry):
            @pl.when(u_rel < npairs)
            def _():
                out_dmas = []
                for rloc in range(2):
                    wbase = (2 * u_rel + rloc) * wrow
                    ho = 2 * u0 + 2 * u_rel + rloc
                    for ck in range(chunks):
                        k = rloc * chunks + ck
                        if k >= 2:
                            out_dmas[k - 2].wait()
                        ob = obufs[k % 2]
                        cbase = (u_rel) * RW + ck * (CW * C)

                        HOIST = 3
                        wvh = [[[wbuf[pl.ds(
                            wbase + (p * KERNEL + a * 6 + b) * L, L)]
                            for b in range(4)] for p in range(2)]
                            for a in range(HOIST)]

                        def jbody(jj, c2, ob=ob, wvh=wvh):
                            for up in range(UNROLL):
                                jp = jj * UNROLL + up
                                ab = 2 * jp * C
                                acc0 = list(bv)
                                acc1 = list(bv)
                                for a in range(5):
                                    off = cbase + a * RW + jp * C
                                    wv = wvh[a] if a < HOIST else \
                                        [[wbuf[pl.ds(
                                            wbase + (p * KERNEL + a * 6 + b) * L, L)]
                                          for b in range(4)] for p in range(2)]
                                    for g in range(NG):
                                        v = [inbuf[pl.ds(off + b * C + L * g, L)]
                                             for b in range(6)]
                                        acc0[g] = (acc0[g] + v[0] * wv[0][0]
                                                   + (v[1] + v[5]) * wv[0][1]
                                                   + (v[2] + v[4]) * wv[0][2]
                                                   + v[3] * wv[0][3])
                                        acc1[g] = (acc1[g] + v[0] * wv[1][0]
                                                   + v[1] * wv[1][1]
                                                   + (v[2] + v[5]) * wv[1][2]
                                                   + (v[3] + v[4]) * wv[1][3])
                                for g in range(NG):
                                    ob[pl.ds(ab + L * g, L)] = acc0[g]
                                    ob[pl.ds(ab + C + L * g, L)] = acc1[g]
                            return c2

                        lax.fori_loop(0, CW // UNROLL, jbody, 0)
                        out_dmas.append(pltpu.async_copy(
                            ob,
                            out_hbm.at[pl.ds(ho * (Wo * C) + ck * (2 * CW * C),
                                             2 * CW * C)],
                            sem_out))
                for d in out_dmas[-2:]:
                    d.wait()
            return carry

        lax.fori_loop(0, max_pairs, row_pair, 0)

    built = []

    def apply(x, bias):
        if not built:
            mesh = plsc.VectorSubcoreMesh(
                core_axis_name="c", subcore_axis_name="s",
                num_cores=NC, num_subcores=NS)
            built.append(pl.kernel(
                layer_body,
                out_type=jax.ShapeDtypeStruct((Ho * Wo * C,), jnp.float32),
                mesh=mesh,
                scratch_types=[
                    pltpu.VMEM((n_slots * RW,), jnp.float32),
                    pltpu.VMEM((2 * CW * C,), jnp.float32),
                    pltpu.VMEM((2 * CW * C,), jnp.float32),
                    pltpu.VMEM((max_pairs * wpair,), jnp.float32),
                    pltpu.VMEM((C,), jnp.float32),
                    pltpu.SemaphoreType.DMA,
                    pltpu.SemaphoreType.DMA,
                ],
            ))
        return built[0](x, jnp.asarray(wsp), bias)

    return apply


_layer1 = _make_layer(0.25, 0.5, 2, 13)
_layer2 = _make_layer(0.5, 1.0, 3, 26)


@jax.jit
def kernel(x, b1, b2):
    xf = x.reshape(-1)
    h = _layer1(xf, b1)
    y = _layer2(h, b2)
    return y.reshape(1, NLAT * NLON, C)


# speedup vs baseline: 1.8013x; 1.0081x over previous
import functools

import jax
import jax.numpy as jnp
import numpy as np
from jax import lax
from jax.experimental import pallas as pl
from jax.experimental.pallas import tpu as pltpu
from jax.experimental.pallas import tpu_sc as plsc

NLAT, NLON, KERNEL = 180, 360, 30
C = 64
L = 16
NG = C // L
NC, NS = 2, 16
NW = NC * NS


def _grid_points(H, W):
    lat = np.pi * (np.arange(H) + 0.5) / H - np.pi / 2.0
    lon = 2.0 * np.pi * np.arange(W) / W
    la, lo = np.meshgrid(lat, lon, indexing='ij')
    pts = np.stack([np.cos(la) * np.cos(lo), np.cos(la) * np.sin(lo),
                    np.sin(la)], axis=-1)
    return pts.reshape(H * W, 3)


def _compressed_weights(in_ratio, out_ratio):
    Hi, Wi = int(round(NLAT * in_ratio)), int(round(NLON * in_ratio))
    Ho, Wo = int(round(NLAT * out_ratio)), int(round(NLON * out_ratio))
    P = _grid_points(Hi, Wi)
    M = _grid_points(Ho, Wo).reshape(Ho, Wo, 3)[:, :2]
    ci = np.clip(((np.arange(Ho) + 0.5) * Hi / Ho).astype(np.int64), 0, Hi - 1)
    cj = (((np.arange(2) + 0.5) * Wi / Wo).astype(np.int64)) % Wi
    di = np.array([-2, -1, 0, 1, 2], dtype=np.int64)
    dj = np.array([-3, -2, -1, 0, 1, 2], dtype=np.int64)
    ii = np.clip(ci[:, None, None, None] + di[None, None, :, None], 0, Hi - 1)
    jj = (cj[None, :, None, None] + dj[None, None, None, :]) % Wi
    idx = (ii * Wi + jj).reshape(Ho, 2, KERNEL)
    dots = np.einsum('hpd,hpkd->hpk', M, P[idx])
    kappa = (Hi * Hi) / 4.0
    w = np.exp(kappa * (dots - 1.0))
    w = w / w.sum(axis=2, keepdims=True)
    return w.astype(np.float32)


def _make_layer(in_ratio, out_ratio, max_pairs, tiles_hi):
    Hi, Wi = int(round(NLAT * in_ratio)), int(round(NLON * in_ratio))
    Ho, Wo = int(round(NLAT * out_ratio)), int(round(NLON * out_ratio))
    assert tiles_hi * max_pairs + (NW - tiles_hi) * (max_pairs - 1) == Ho // 2
    n_slots = max_pairs + 4
    BW = Wi + 5
    RW = BW * C
    wrow = 2 * KERNEL * L

    chunks = Wo // 180
    CW = Wo // (2 * chunks)
    UNROLL = 1
    wpair = 2 * wrow

    wc = _compressed_weights(in_ratio, out_ratio)
    wsp = np.repeat(wc.reshape(Ho, 2 * KERNEL), L, axis=1)
    wsp = np.concatenate([wsp, np.zeros((2, wsp.shape[1]), np.float32)])
    wsp = wsp.reshape(-1)

    def layer_body(x_hbm, w_hbm, b_hbm, out_hbm,
                   inbuf, obuf0, obuf1, wbuf, bbuf, sem_in, sem_out):
        wid = lax.axis_index("s") * NC + lax.axis_index("c")
        npairs = jnp.where(wid < tiles_hi, max_pairs, max_pairs - 1)
        u0 = jnp.where(wid < tiles_hi, max_pairs * wid,
                       max_pairs * tiles_hi + (max_pairs - 1) * (wid - tiles_hi))

        stage = [pltpu.async_copy(b_hbm, bbuf, sem_in),
                 pltpu.async_copy(
                     w_hbm.at[pl.ds(u0 * wpair, max_pairs * wpair)],
                     wbuf, sem_in)]
        for r in range(n_slots):
            i_r = jnp.clip(u0 - 2 + r, 0, Hi - 1)
            src = i_r * (Wi * C)
            dst = r * RW
            stage.append(pltpu.async_copy(
                x_hbm.at[pl.ds(src, Wi * C)],
                inbuf.at[pl.ds(dst + 3 * C, Wi * C)], sem_in))
            stage.append(pltpu.async_copy(
                x_hbm.at[pl.ds(src + (Wi - 3) * C, 3 * C)],
                inbuf.at[pl.ds(dst, 3 * C)], sem_in))
            stage.append(pltpu.async_copy(
                x_hbm.at[pl.ds(src, 2 * C)],
                inbuf.at[pl.ds(dst + (3 + Wi) * C, 2 * C)], sem_in))
        for h in stage:
            h.wait()

        bv = [bbuf[pl.ds(L * g, L)] for g in range(NG)]
        obufs = [obuf0, obuf1]

        def row_pair(u_rel, carry):
            @pl.when(u_rel < npairs)
            def _():
                out_dmas = []
                for rloc in range(2):
                    wbase = (2 * u_rel + rloc) * wrow
                    ho = 2 * u0 + 2 * u_rel + rloc
                    for ck in range(chunks):
                        k = rloc * chunks + ck
                        if k >= 2:
                            out_dmas[k - 2].wait()
                        ob = obufs[k % 2]
                        cbase = (u_rel) * RW + ck * (CW * C)

                        HOIST = 4
                        wvh = [[[wbuf[pl.ds(
                            wbase + (p * KERNEL + a * 6 + b) * L, L)]
                            for b in range(4)] for p in range(2)]
                            for a in range(HOIST)]

                        def jbody(jj, c2, ob=ob, wvh=wvh):
                            for up in range(UNROLL):
                                jp = jj * UNROLL + up
                                ab = 2 * jp * C
                                acc0 = list(bv)
                                acc1 = list(bv)
                                for a in range(5):
                                    off = cbase + a * RW + jp * C
                                    wv = wvh[a] if a < HOIST else \
                                        [[wbuf[pl.ds(
                                            wbase + (p * KERNEL + a * 6 + b) * L, L)]
                                          for b in range(4)] for p in range(2)]
                                    for g in range(NG):
                                        v = [inbuf[pl.ds(off + b * C + L * g, L)]
                                             for b in range(6)]
                                        acc0[g] = (acc0[g] + v[0] * wv[0][0]
                                                   + (v[1] + v[5]) * wv[0][1]
                                                   + (v[2] + v[4]) * wv[0][2]
                                                   + v[3] * wv[0][3])
                                        acc1[g] = (acc1[g] + v[0] * wv[1][0]
                                                   + v[1] * wv[1][1]
                                                   + (v[2] + v[5]) * wv[1][2]
                                                   + (v[3] + v[4]) * wv[1][3])
                                for g in range(NG):
                                    ob[pl.ds(ab + L * g, L)] = acc0[g]
                                    ob[pl.ds(ab + C + L * g, L)] = acc1[g]
                            return c2

                        lax.fori_loop(0, CW // UNROLL, jbody, 0)
                        out_dmas.append(pltpu.async_copy(
                            ob,
                            out_hbm.at[pl.ds(ho * (Wo * C) + ck * (2 * CW * C),
                                             2 * CW * C)],
                            sem_out))
                for d in out_dmas[-2:]:
                    d.wait()
            return carry

        lax.fori_loop(0, max_pairs, row_pair, 0)

    built = []

    def apply(x, bias):
        if not built:
            mesh = plsc.VectorSubcoreMesh(
                core_axis_name="c", subcore_axis_name="s",
                num_cores=NC, num_subcores=NS)
            built.append(pl.kernel(
                layer_body,
                out_type=jax.ShapeDtypeStruct((Ho * Wo * C,), jnp.float32),
                mesh=mesh,
                scratch_types=[
                    pltpu.VMEM((n_slots * RW,), jnp.float32),
                    pltpu.VMEM((2 * CW * C,), jnp.float32),
                    pltpu.VMEM((2 * CW * C,), jnp.float32),
                    pltpu.VMEM((max_pairs * wpair,), jnp.float32),
                    pltpu.VMEM((C,), jnp.float32),
                    pltpu.SemaphoreType.DMA,
                    pltpu.SemaphoreType.DMA,
                ],
            ))
        return built[0](x, jnp.asarray(wsp), bias)

    return apply


_layer1 = _make_layer(0.25, 0.5, 2, 13)
_layer2 = _make_layer(0.5, 1.0, 3, 26)


@jax.jit
def kernel(x, b1, b2):
    xf = x.reshape(-1)
    h = _layer1(xf, b1)
    y = _layer2(h, b2)
    return y.reshape(1, NLAT * NLON, C)
